# baseline (speedup 1.0000x reference)
import jax
import jax.numpy as jnp
from jax.experimental import pallas as pl
from jax.experimental.pallas import tpu as pltpu

CLS_NUM = 1000
D = 512
N_PREFIX = 3
N_SUFFIX = 2
N_CTX = N_PREFIX + N_SUFFIX
N_CLS_TOK = 3
CTX_LEN = 77
PAD_SIZE = CTX_LEN - (N_CTX + N_CLS_TOK + 2)
PAD_LEN = 75

G = 125
NG = CLS_NUM // G


def _body(ctx_ref, sot_ref, eot_ref, pad_ref, cls_hbm, out_hbm,
          t0, tmpl, tmpl_sem, cls_sem):
    t0[0:1, :] = sot_ref[0, :, :]
    t0[1:4, :] = ctx_ref[0, 0:N_PREFIX, :]
    t0[4:7, :] = ctx_ref[0, 0:N_CLS_TOK, :]
    t0[7:9, :] = ctx_ref[0, N_PREFIX:, :]
    t0[9:10, :] = eot_ref[0, :, :]
    t0[10:, :] = pad_ref[0, 0:PAD_SIZE, :]
    tmpl[:, :, :] = jnp.broadcast_to(t0[:, :][None], (G, CTX_LEN, D))

    def tmpl_copy(g):
        return pltpu.make_async_copy(
            tmpl, out_hbm.at[pl.ds(g * G, G)], tmpl_sem.at[g]
        )

    def cls_copy(g):
        return pltpu.make_async_copy(
            cls_hbm.at[pl.ds(g * G, G)],
            out_hbm.at[pl.ds(g * G, G), pl.ds(4, N_CLS_TOK)],
            cls_sem.at[g],
        )

    tmpl_copy(0).start()
    for g in range(NG):
        tmpl_copy(g).wait()
        if g + 1 < NG:
            tmpl_copy(g + 1).start()
        cls_copy(g).start()
    for g in range(NG):
        cls_copy(g).wait()


def kernel(ctx, emb_sot, emb_cls, emb_eot, emb_pad):
    return pl.pallas_call(
        _body,
        in_specs=[
            pl.BlockSpec(memory_space=pltpu.MemorySpace.VMEM),
            pl.BlockSpec(memory_space=pltpu.MemorySpace.VMEM),
            pl.BlockSpec(memory_space=pltpu.MemorySpace.VMEM),
            pl.BlockSpec(memory_space=pltpu.MemorySpace.VMEM),
            pl.BlockSpec(memory_space=pltpu.MemorySpace.HBM),
        ],
        out_specs=pl.BlockSpec(memory_space=pltpu.MemorySpace.HBM),
        out_shape=jax.ShapeDtypeStruct((CLS_NUM, CTX_LEN, D), jnp.float32),
        scratch_shapes=[
            pltpu.VMEM((CTX_LEN, D), jnp.float32),
            pltpu.VMEM((G, CTX_LEN, D), jnp.float32),
            pltpu.SemaphoreType.DMA((NG,)),
            pltpu.SemaphoreType.DMA((NG,)),
        ],
    )(ctx, emb_sot, emb_eot, emb_pad, emb_cls)


# speedup vs baseline: 1.0054x; 1.0054x over previous
import jax
import jax.numpy as jnp
from jax.experimental import pallas as pl
from jax.experimental.pallas import tpu as pltpu

CLS_NUM = 1000
D = 512
N_PREFIX = 3
N_SUFFIX = 2
N_CTX = N_PREFIX + N_SUFFIX
N_CLS_TOK = 3
CTX_LEN = 77
PAD_SIZE = CTX_LEN - (N_CTX + N_CLS_TOK + 2)
PAD_LEN = 75

NG = 10
G = CLS_NUM // NG


def _body(ctx_ref, sot_ref, eot_ref, pad_ref, cls_hbm, out_hbm,
          t0, tmpl, tmpl_sem, cls_sem):
    t0[0:1, :] = sot_ref[0, :, :]
    t0[1:4, :] = ctx_ref[0, 0:N_PREFIX, :]
    t0[4:7, :] = ctx_ref[0, 0:N_CLS_TOK, :]
    t0[7:9, :] = ctx_ref[0, N_PREFIX:, :]
    t0[9:10, :] = eot_ref[0, :, :]
    t0[10:, :] = pad_ref[0, 0:PAD_SIZE, :]
    tmpl[:, :, :] = jnp.broadcast_to(t0[:, :][None], (G, CTX_LEN, D))

    def tmpl_copy(g):
        return pltpu.make_async_copy(
            tmpl, out_hbm.at[pl.ds(g * G, G)], tmpl_sem.at[g]
        )

    def cls_copy(g):
        return pltpu.make_async_copy(
            cls_hbm.at[pl.ds(g * G, G)],
            out_hbm.at[pl.ds(g * G, G), pl.ds(4, N_CLS_TOK)],
            cls_sem.at[g],
        )

    for g in range(NG):
        tmpl_copy(g).start()
    for g in range(NG):
        tmpl_copy(g).wait()
        cls_copy(g).start()
    for g in range(NG):
        cls_copy(g).wait()


def kernel(ctx, emb_sot, emb_cls, emb_eot, emb_pad):
    return pl.pallas_call(
        _body,
        in_specs=[
            pl.BlockSpec(memory_space=pltpu.MemorySpace.VMEM),
            pl.BlockSpec(memory_space=pltpu.MemorySpace.VMEM),
            pl.BlockSpec(memory_space=pltpu.MemorySpace.VMEM),
            pl.BlockSpec(memory_space=pltpu.MemorySpace.VMEM),
            pl.BlockSpec(memory_space=pltpu.MemorySpace.HBM),
        ],
        out_specs=pl.BlockSpec(memory_space=pltpu.MemorySpace.HBM),
        out_shape=jax.ShapeDtypeStruct((CLS_NUM, CTX_LEN, D), jnp.float32),
        scratch_shapes=[
            pltpu.VMEM((CTX_LEN, D), jnp.float32),
            pltpu.VMEM((G, CTX_LEN, D), jnp.float32),
            pltpu.SemaphoreType.DMA((NG,)),
            pltpu.SemaphoreType.DMA((NG,)),
        ],
    )(ctx, emb_sot, emb_eot, emb_pad, emb_cls)


# speedup vs baseline: 1.8283x; 1.8185x over previous
import jax
import jax.numpy as jnp
from jax.experimental import pallas as pl
from jax.experimental.pallas import tpu as pltpu

CLS_NUM = 1000
D = 512
N_PREFIX = 3
N_SUFFIX = 2
N_CTX = N_PREFIX + N_SUFFIX
N_CLS_TOK = 3
CTX_LEN = 77
PAD_SIZE = CTX_LEN - (N_CTX + N_CLS_TOK + 2)
PAD_LEN = 75

B = 40
NSTEPS = CLS_NUM // B


def _body(ctx_ref, sot_ref, eot_ref, pad_ref, cls_ref, out_ref, t0, tmpl):
    @pl.when(pl.program_id(0) == 0)
    def _build():
        t0[0:1, :] = sot_ref[0, :, :]
        t0[1:4, :] = ctx_ref[0, 0:N_PREFIX, :]
        t0[4:7, :] = ctx_ref[0, 0:N_CLS_TOK, :]
        t0[7:9, :] = ctx_ref[0, N_PREFIX:, :]
        t0[9:10, :] = eot_ref[0, :, :]
        t0[10:, :] = pad_ref[0, 0:PAD_SIZE, :]
        tmpl[:, :, :] = jnp.broadcast_to(t0[:, :][None], (B, CTX_LEN, D))

    out_ref[:, :, :] = tmpl[:, :, :]
    out_ref[:, 4:7, :] = cls_ref[:, :, :]


def kernel(ctx, emb_sot, emb_cls, emb_eot, emb_pad):
    return pl.pallas_call(
        _body,
        grid=(NSTEPS,),
        in_specs=[
            pl.BlockSpec((1, N_CTX, D), lambda i: (0, 0, 0)),
            pl.BlockSpec((1, 1, D), lambda i: (0, 0, 0)),
            pl.BlockSpec((1, 1, D), lambda i: (0, 0, 0)),
            pl.BlockSpec((1, PAD_LEN, D), lambda i: (0, 0, 0)),
            pl.BlockSpec((B, N_CLS_TOK, D), lambda i: (i, 0, 0)),
        ],
        out_specs=pl.BlockSpec((B, CTX_LEN, D), lambda i: (i, 0, 0)),
        out_shape=jax.ShapeDtypeStruct((CLS_NUM, CTX_LEN, D), jnp.float32),
        scratch_shapes=[
            pltpu.VMEM((CTX_LEN, D), jnp.float32),
            pltpu.VMEM((B, CTX_LEN, D), jnp.float32),
        ],
    )(ctx, emb_sot, emb_eot, emb_pad, emb_cls)
